# Initial kernel scaffold; baseline (speedup 1.0000x reference)
import jax
import jax.numpy as jnp
from jax.experimental import pallas as pl


def kernel(scores, bbox_deltas, image_metadata):
    raise NotImplementedError("write your pallas kernel here")


# speedup vs baseline: 18.9337x; 18.9337x over previous
import jax
import jax.numpy as jnp
from jax.experimental import pallas as pl
from jax.experimental.pallas import tpu as pltpu

_H = 48
_W = 48
_A = 9
_N = _A * _H * _W
_ROWS = _N // 128
_K = 1000
_KP = 1024
_POST = 300
_OUTP = 304
_THRESH = 0.7
_MIN_SIZE = 2.0

_SZ_W = (4.0, 8.0, 16.0, 4.0, 8.0, 8.0, 16.0, 16.0, 32.0)
_SZ_H = (4.0, 8.0, 16.0, 8.0, 4.0, 16.0, 8.0, 32.0, 16.0)


def _box_kernel(sc_ref, dx_ref, dy_ref, dw_ref, dh_ref,
                x1_ref, y1_ref, w_ref, h_ref, x2n_ref, y2n_ref, ar_ref, fs_ref):
    b = (jax.lax.broadcasted_iota(jnp.int32, (_ROWS, 128), 0) * 128
         + jax.lax.broadcasted_iota(jnp.int32, (_ROWS, 128), 1))
    a = b // (_H * _W)
    rem = b - a * (_H * _W)
    yi = rem // _W
    xi = rem - yi * _W

    wsz = jnp.full((_ROWS, 128), _SZ_W[0], jnp.float32)
    hsz = jnp.full((_ROWS, 128), _SZ_H[0], jnp.float32)
    for k in range(1, _A):
        m = a == k
        wsz = jnp.where(m, _SZ_W[k], wsz)
        hsz = jnp.where(m, _SZ_H[k], hsz)

    anc_x = xi.astype(jnp.float32) + 0.5 - wsz / 2.0
    anc_y = yi.astype(jnp.float32) + 0.5 - hsz / 2.0

    ax = jnp.maximum(anc_x + dx_ref[...], 0.0)
    ay = jnp.maximum(anc_y + dy_ref[...], 0.0)
    aw = jnp.maximum(wsz + dw_ref[...], 0.0)
    ah = jnp.maximum(hsz + dh_ref[...], 0.0)

    x2r = ax + aw
    y2r = ay + ah
    x1 = jnp.minimum(ax, float(_H))
    y1 = jnp.minimum(ay, float(_W))
    x2 = jnp.minimum(x2r, float(_H))
    y2 = jnp.minimum(y2r, float(_W))
    w = x2 - x1
    h = y2 - y1

    keep = (w >= _MIN_SIZE) & (h >= _MIN_SIZE)
    x1_ref[...] = x1
    y1_ref[...] = y1
    w_ref[...] = w
    h_ref[...] = h
    x2n_ref[...] = x1 + w
    y2n_ref[...] = y1 + h
    ar_ref[...] = w * h
    fs_ref[...] = jnp.where(keep, sc_ref[...], -jnp.inf)


def _nms_kernel(rows_ref, cols_ref, pprop_ref, out_ref, ov_scr):
    cx1 = cols_ref[0:1, :]
    cy1 = cols_ref[1:2, :]
    cx2 = cols_ref[2:3, :]
    cy2 = cols_ref[3:4, :]
    car = cols_ref[4:5, :]
    csc = cols_ref[5:6, :]

    for bi in range(_KP // 128):
        blk = rows_ref[bi * 128:(bi + 1) * 128, :]
        rx1 = blk[:, 0:1]
        ry1 = blk[:, 1:2]
        rx2 = blk[:, 2:3]
        ry2 = blk[:, 3:4]
        xx1 = jnp.maximum(rx1, cx1)
        yy1 = jnp.maximum(ry1, cy1)
        xx2 = jnp.minimum(rx2, cx2)
        yy2 = jnp.minimum(ry2, cy2)
        wm = jnp.maximum(xx2 - xx1 + 1.0, 0.0)
        hm = jnp.maximum(yy2 - yy1 + 1.0, 0.0)
        ov_scr[bi * 128:(bi + 1) * 128, :] = (wm * hm) / jnp.maximum(car, 1e-6)

    kmf0 = jnp.where(jnp.isfinite(csc), 1.0, 0.0)
    lane = jax.lax.broadcasted_iota(jnp.int32, (1, _KP), 1)

    def body(i, kmf):
        row = ov_scr[pl.ds(i, 1), :]
        alive = jnp.sum(jnp.where(lane == i, kmf, 0.0)) > 0.0
        supf = jnp.where((row >= _THRESH) & alive & (lane != i), 1.0, 0.0)
        return kmf * (1.0 - supf)

    kmf = jax.lax.fori_loop(0, _KP, body, kmf0)

    c = kmf
    s = 1
    while s < _KP:
        c = c + jnp.concatenate(
            [jnp.zeros((1, s), jnp.float32), c[:, :_KP - s]], axis=1)
        s *= 2
    rank = c - 1.0
    sel = (kmf > 0.0) & (rank < float(_POST))

    kidx = jax.lax.broadcasted_iota(jnp.int32, (_OUTP, _KP), 0)
    onehot = jnp.where((kidx == rank.astype(jnp.int32)) & sel, 1.0, 0.0)
    out_ref[...] = jnp.dot(onehot, pprop_ref[...],
                           preferred_element_type=jnp.float32)


def kernel(scores, bbox_deltas, image_metadata):
    f32 = jnp.float32
    sc = scores.reshape(_ROWS, 128)
    d = bbox_deltas.reshape(_A, _H, _W, 4)
    dx = d[..., 0].reshape(_ROWS, 128)
    dy = d[..., 1].reshape(_ROWS, 128)
    dw = d[..., 2].reshape(_ROWS, 128)
    dh = d[..., 3].reshape(_ROWS, 128)

    shp = jax.ShapeDtypeStruct((_ROWS, 128), f32)
    x1, y1, w, h, x2n, y2n, ar, fs = pl.pallas_call(
        _box_kernel,
        out_shape=[shp] * 8,
    )(sc, dx, dy, dw, dh)

    fs_flat = fs.reshape(-1)
    top_sc, order = jax.lax.top_k(fs_flat, _K)
    g = lambda arr: arr.reshape(-1)[order]
    px1, py1, pw, ph, px2, py2, par = (g(x1), g(y1), g(w), g(h),
                                       g(x2n), g(y2n), g(ar))

    perm = jnp.argsort(py2)[::-1]
    sx1 = px1[perm]
    sy1 = py1[perm]
    sx2 = px2[perm]
    sy2 = py2[perm]
    sar = par[perm]
    ssc = top_sc[perm]
    pprop = jnp.stack([px1, py1, pw, ph], axis=1)[perm]

    pad = _KP - _K
    z = jnp.zeros((pad,), f32)
    sx1 = jnp.concatenate([sx1, z])
    sy1 = jnp.concatenate([sy1, z])
    sx2 = jnp.concatenate([sx2, z])
    sy2 = jnp.concatenate([sy2, z])
    sar = jnp.concatenate([sar, z])
    ssc = jnp.concatenate([ssc, jnp.full((pad,), -jnp.inf, f32)])
    zv = jnp.zeros((_KP,), f32)
    rows = jnp.stack([sx1, sy1, sx2, sy2, sar, ssc, zv, zv], axis=1)
    cols = rows.T
    pprop = jnp.concatenate([pprop, jnp.zeros((pad, 4), f32)], axis=0)

    out = pl.pallas_call(
        _nms_kernel,
        out_shape=jax.ShapeDtypeStruct((_OUTP, 4), f32),
        scratch_shapes=[pltpu.VMEM((_KP, _KP), f32)],
    )(rows, cols, pprop)

    return out[:_POST][None]
